# baseline (speedup 1.0000x reference)
import functools

import jax
import jax.numpy as jnp
from jax.experimental import pallas as pl
from jax.experimental.pallas import tpu as pltpu

_EPS = 1e-8
_NS = 4


def _part_t(x, ca_unit_t_b16, lab_row, hi):
    raw = jnp.dot(
        x.astype(jnp.bfloat16), ca_unit_t_b16, preferred_element_type=jnp.float32
    )
    raw_t = raw.T
    sumsq = jnp.sum(x * x, axis=1, keepdims=True)
    inv = jax.lax.rsqrt(jnp.maximum(sumsq, _EPS * _EPS)).reshape(1, -1)
    cos_t = raw_t * inv
    k, mt = cos_t.shape
    kidx = jax.lax.broadcasted_iota(jnp.int32, (k, mt), 0)
    uninit = jnp.where(lab_row == kidx, jnp.float32(1.0), jnp.float32(-1.0))
    return jnp.where(hi > 0.0, uninit, cos_t)


def _simrel_tile(ca_ref, *refs):
    x_refs = refs[:_NS]
    lab_ref, out_ref, ca_unit_ref, hi_ref = refs[_NS:]
    nj = pl.num_programs(0) // lab_ref.shape[0]
    b = pl.program_id(0) // nj

    @pl.when(pl.program_id(0) == 0)
    def _prep():
        ca = ca_ref[...]
        ca_sq = jnp.sum(ca * ca, axis=1, keepdims=True)
        ca_norm = jnp.sqrt(ca_sq)
        ca_unit = ca / jnp.maximum(ca_norm, _EPS)
        ca_unit_ref[...] = ca_unit.T.astype(jnp.bfloat16)
        has_inf = jnp.any(jnp.isinf(ca), axis=1, keepdims=True)
        hi_ref[...] = has_inf.astype(jnp.float32)

    ca_unit_t_b16 = ca_unit_ref[...]
    hi = hi_ref[...]
    mt = x_refs[0].shape[0]
    lab = lab_ref[pl.ds(b, 1), :]
    for q, x_ref in enumerate(x_refs):
        out_ref[0, :, q * mt : (q + 1) * mt] = _part_t(
            x_ref[...], ca_unit_t_b16, lab[:, q * mt : (q + 1) * mt], hi
        )


@functools.partial(jax.jit, static_argnames=())
def kernel(inputs, labels, class_avgs):
    b, t, d = inputs.shape
    k = class_avgs.shape[0]
    m = b * t
    mt = 1024
    sup = _NS * mt
    nj = t // sup
    n_tiles = m // sup

    x2 = inputs.reshape(m, d)
    lab = labels.astype(jnp.int32)

    out_bkt = pl.pallas_call(
        _simrel_tile,
        grid=(n_tiles,),
        in_specs=[pl.BlockSpec((k, d), lambda i: (0, 0))]
        + [
            pl.BlockSpec((mt, d), lambda i, q=q: (_NS * i + q, 0))
            for q in range(_NS)
        ]
        + [pl.BlockSpec((b, sup), lambda i, nj=nj: (0, i % nj))],
        out_specs=pl.BlockSpec((1, k, sup), lambda i, nj=nj: (i // nj, 0, i % nj)),
        out_shape=jax.ShapeDtypeStruct((b, k, t), jnp.float32),
        scratch_shapes=[
            pltpu.VMEM((d, k), jnp.bfloat16),
            pltpu.VMEM((k, 1), jnp.float32),
        ],
        compiler_params=pltpu.CompilerParams(
            dimension_semantics=("arbitrary",),
        ),
    )(class_avgs, *([x2] * _NS), lab)
    return jnp.transpose(out_bkt, (0, 2, 1))
